# baseline (speedup 1.0000x reference)
import functools

import jax
import jax.numpy as jnp
from jax import lax
from jax.experimental import pallas as pl
from jax.experimental.pallas import tpu as pltpu
from jax.experimental.pallas import tpu_sc as plsc

_NEG = 200
_CI = 200
_D = 128
_NCHUNK = _D // 16
_CI_TILES = 13
_NEG_PER_SC = _NEG // 2
_LN2 = 0.6931471805599453
_SQRT1_2 = 0.7071067811865476


def _vsig(x):
    return 1.0 / (1.0 + jnp.exp(-x))


def _vlog(x):
    z = (x - 1.0) / (x + 1.0)
    z2 = z * z
    p = 1.0 / 25.0
    for k in (23, 21, 19, 17, 15, 13, 11, 9, 7, 5, 3, 1):
        p = p * z2 + 1.0 / k
    return 2.0 * z * p


def _bc(s):
    return jnp.broadcast_to(s, (16,))


_GDN = lax.GatherDimensionNumbers(
    offset_dims=(), collapsed_slice_dims=(0,), start_index_map=(0,))


def _lane_perm(v, idx):
    return lax.gather(v, idx[:, None], _GDN, slice_sizes=(1,),
                      mode=lax.GatherScatterMode.PROMISE_IN_BOUNDS)


def _hsum(v):
    idx = lax.iota(jnp.int32, 16)
    for k in (8, 4, 2, 1):
        v = v + _lane_perm(v, jnp.bitwise_xor(idx, k))
    return v


@functools.cache
def _make_sc_loss():
    @functools.partial(
        pl.kernel,
        out_type=[jax.ShapeDtypeStruct((2, 16), jnp.float32),
                  jax.ShapeDtypeStruct((32, 16), jnp.float32)],
        mesh=plsc.VectorSubcoreMesh(core_axis_name="c", subcore_axis_name="s"),
        scratch_types=[
            pltpu.VMEM((16,), jnp.int32),
            pltpu.VMEM((8,), jnp.int32),
            pltpu.VMEM((16, _D), jnp.float32),
            pltpu.VMEM((8, _D), jnp.float32),
            pltpu.VMEM((1, _D), jnp.float32),
            pltpu.VMEM((16, _D), jnp.float32),
            pltpu.VMEM((1, 16), jnp.float32),
            pltpu.VMEM((16, 16), jnp.float32),
            pltpu.VMEM_SHARED((16, _D), jnp.float32),
            pltpu.SemaphoreType.DMA,
            pltpu.SemaphoreType.DMA,
        ],
    )
    def _sc_loss(cipad_hbm, negpad_hbm, aux_hbm, up_hbm, pp_hbm,
                 out_hbm, stage_hbm,
                 cidx, nidx, crows, nrows, prow, pall, wbuf, wall,
                 sh_part, sem1, sem2):
        c = lax.axis_index("c")
        s = lax.axis_index("s")

        @pl.when(s < _CI_TILES)
        def _():
            pltpu.sync_copy(cipad_hbm.at[pl.ds(s * 16, 16)], cidx)
            pltpu.sync_copy(negpad_hbm.at[pl.ds(c * 128 + s * 8, 8)], nidx)
            gc = pltpu.async_copy(pp_hbm.at[cidx], crows, sem1)
            gn = pltpu.async_copy(pp_hbm.at[nidx], nrows, sem2)
            gc.wait()
            gn.wait()
            hi_w = _bc(jnp.where(s < 12, 1.0, 0.0).astype(jnp.float32))
            for ch in range(_NCHUNK):
                sl = pl.ds(ch * 16, 16)
                lo = crows[0, sl]
                for r in range(1, 8):
                    lo = lo + crows[r, sl]
                hi = crows[8, sl]
                for r in range(9, 16):
                    hi = hi + crows[r, sl]
                prow[0, sl] = lo + hi * hi_w
            pltpu.sync_copy(prow, sh_part.at[pl.ds(s, 1)])

        @pl.when(s == 13)
        def _():
            pltpu.sync_copy(aux_hbm.at[pl.ds(0, 8)], nidx)
            pltpu.async_copy(pp_hbm.at[nidx], nrows, sem2).wait()
            pltpu.sync_copy(nrows.at[pl.ds(0, 1)], sh_part.at[pl.ds(13, 1)])

        @pl.when(s == 14)
        def _():
            pltpu.sync_copy(aux_hbm.at[pl.ds(8, 8)], nidx)
            pltpu.async_copy(up_hbm.at[nidx], nrows, sem2).wait()
            pltpu.sync_copy(nrows.at[pl.ds(0, 1)], sh_part.at[pl.ds(14, 1)])

        plsc.subcore_barrier()

        wbuf[0, :] = jnp.zeros((16,), jnp.float32)

        @pl.when(s < _CI_TILES)
        def _():
            pltpu.sync_copy(sh_part, pall)
            csum = []
            uvec = []
            for ch in range(_NCHUNK):
                sl = pl.ds(ch * 16, 16)
                a = pall[0, sl]
                for r in range(1, _CI_TILES):
                    a = a + pall[r, sl]
                csum.append(a)
                uvec.append(pall[14, sl])
            wacc = jnp.zeros((16,), jnp.float32)
            for r in range(8):
                sl0 = pl.ds(0, 16)
                row0 = nrows[r, sl0]
                pa = row0 * uvec[0]
                pb = row0 * csum[0]
                for ch in range(1, _NCHUNK):
                    sl = pl.ds(ch * 16, 16)
                    rowc = nrows[r, sl]
                    pa = pa + rowc * uvec[ch]
                    pb = pb + rowc * csum[ch]
                av = _hsum(pa)
                bv = _hsum(pb) / float(_CI)
                w = _vlog(1.0 - _vsig(av) * _vsig(bv))
                valid = _bc(jnp.where(s * 8 + r < _NEG_PER_SC, 1.0, 0.0)
                            .astype(jnp.float32))
                wacc = wacc + w * valid
            wbuf[0, :] = wacc

        @pl.when(jnp.logical_and(s == 13, c == 0))
        def _():
            pltpu.sync_copy(sh_part, pall)
            sacc = pall[13, pl.ds(0, 16)] * pall[14, pl.ds(0, 16)]
            tacc0 = pall[0, pl.ds(0, 16)]
            for r in range(1, _CI_TILES):
                tacc0 = tacc0 + pall[r, pl.ds(0, 16)]
            tacc = tacc0 * pall[13, pl.ds(0, 16)]
            for ch in range(1, _NCHUNK):
                sl = pl.ds(ch * 16, 16)
                pivec = pall[13, sl]
                sacc = sacc + pivec * pall[14, sl]
                cs = pall[0, sl]
                for r in range(1, _CI_TILES):
                    cs = cs + pall[r, sl]
                tacc = tacc + cs * pivec
            sv = _hsum(sacc)
            tv = _hsum(tacc) / float(_CI)
            wbuf[0, :] = _vlog(_vsig(sv)) + _vlog(_vsig(tv))

        pltpu.sync_copy(wbuf, stage_hbm.at[pl.ds(c * 16 + s, 1)])
        plsc.subcore_barrier()

        @pl.when(s == 0)
        def _():
            pltpu.sync_copy(stage_hbm.at[pl.ds(c * 16, 16)], wall)
            tot = wall[0, :]
            for r in range(1, 16):
                tot = tot + wall[r, :]
            wbuf[0, :] = tot
            pltpu.sync_copy(wbuf, out_hbm.at[pl.ds(c, 1)])

    return _sc_loss


def kernel(userid, poii, Ci, neg_p, UserPreference, PoiPreference):
    ci = Ci.astype(jnp.int32)
    neg = neg_p.astype(jnp.int32)
    cipad = jnp.zeros((16 * _CI_TILES,), jnp.int32).at[:_CI].set(ci)
    negpad = (jnp.zeros((256,), jnp.int32)
              .at[0:100].set(neg[0:100])
              .at[128:228].set(neg[100:200]))
    aux = jnp.concatenate(
        [jnp.broadcast_to(poii.astype(jnp.int32), (8,)),
         jnp.broadcast_to(userid.astype(jnp.int32), (8,))])
    part, _ = _make_sc_loss()(cipad, negpad, aux, UserPreference, PoiPreference)
    return -(part[0, 0] + part[1, 0])


# speedup vs baseline: 1.2230x; 1.2230x over previous
import functools

import jax
import jax.numpy as jnp
from jax import lax
from jax.experimental import pallas as pl
from jax.experimental.pallas import tpu as pltpu
from jax.experimental.pallas import tpu_sc as plsc

_NEG = 200
_CI = 200
_D = 128
_ROWS_PER_TILE = 8
_NEG_TILES = _NEG // _ROWS_PER_TILE
_PI_TILE = _NEG_TILES
_U_TILE = _NEG_TILES + 1
_CSUM_BASE = _NEG + _ROWS_PER_TILE
_U_ROW = _CSUM_BASE + _NEG_TILES
_PACK_ROWS = 240


@functools.cache
def _make_sc_gather():
    @functools.partial(
        pl.kernel,
        out_type=jax.ShapeDtypeStruct((_PACK_ROWS, _D), jnp.float32),
        mesh=plsc.VectorSubcoreMesh(core_axis_name="c", subcore_axis_name="s"),
        scratch_types=[
            pltpu.VMEM((16,), jnp.int32),
            pltpu.VMEM((16, _D), jnp.float32),
            pltpu.VMEM((1, _D), jnp.float32),
            pltpu.SemaphoreType.DMA,
            pltpu.SemaphoreType.DMA,
        ],
    )
    def _sc_gather(pidx_hbm, up_hbm, pp_hbm, out_hbm,
                   idx16, rows16, srow, sem1, sem2):
        wid = lax.axis_index("s") * 2 + lax.axis_index("c")

        @pl.when(wid < _NEG_TILES)
        def _():
            base = wid * _ROWS_PER_TILE
            pltpu.sync_copy(pidx_hbm.at[pl.ds(wid * 16, 16)], idx16)
            pltpu.async_copy(pp_hbm.at[idx16], rows16, sem1).wait()
            w1 = pltpu.async_copy(rows16.at[pl.ds(0, _ROWS_PER_TILE)],
                                  out_hbm.at[pl.ds(base, _ROWS_PER_TILE)], sem2)
            for c in range(_D // 16):
                sl = pl.ds(c * 16, 16)
                acc = rows16[8, sl]
                for r in range(9, 16):
                    acc = acc + rows16[r, sl]
                srow[0, sl] = acc
            w2 = pltpu.async_copy(srow, out_hbm.at[pl.ds(_CSUM_BASE + wid, 1)],
                                  sem1)
            w1.wait()
            w2.wait()

        @pl.when(wid == _PI_TILE)
        def _():
            pltpu.sync_copy(pidx_hbm.at[pl.ds(400, 8)], idx16.at[pl.ds(0, 8)])
            pltpu.async_copy(pp_hbm.at[idx16.at[pl.ds(0, 8)]],
                             rows16.at[pl.ds(0, 8)], sem1).wait()
            pltpu.sync_copy(rows16.at[pl.ds(0, 1)], out_hbm.at[pl.ds(_NEG, 1)])

        @pl.when(wid == _U_TILE)
        def _():
            pltpu.sync_copy(pidx_hbm.at[pl.ds(408, 8)], idx16.at[pl.ds(0, 8)])
            pltpu.async_copy(up_hbm.at[idx16.at[pl.ds(0, 8)]],
                             rows16.at[pl.ds(0, 8)], sem1).wait()
            pltpu.sync_copy(rows16.at[pl.ds(0, 1)], out_hbm.at[pl.ds(_U_ROW, 1)])

    return _sc_gather


def _finish_body(x_ref, o_ref):
    neg = x_ref[0:_NEG, :]
    pi = x_ref[_NEG:_NEG + 1, :]
    u = x_ref[_U_ROW:_U_ROW + 1, :]
    csum = jnp.sum(x_ref[_CSUM_BASE:_CSUM_BASE + _NEG_TILES, :],
                   axis=0, keepdims=True)

    s = jnp.sum(u * pi)
    t = jnp.sum(csum * pi) / float(_CI)
    a = jnp.sum(neg * u, axis=1, keepdims=True)
    b = jnp.sum(neg * csum, axis=1, keepdims=True) / float(_CI)

    score = jax.nn.sigmoid(s) * jax.nn.sigmoid(t)
    neg_score = jax.nn.sigmoid(a) * jax.nn.sigmoid(b)
    loss = -(jnp.log(score) + jnp.sum(jnp.log(1.0 - neg_score)))
    o_ref[0, 0] = loss


_finish = pl.pallas_call(
    _finish_body,
    out_shape=jax.ShapeDtypeStruct((1, 1), jnp.float32),
    out_specs=pl.BlockSpec(memory_space=pltpu.SMEM),
)


def kernel(userid, poii, Ci, neg_p, UserPreference, PoiPreference):
    neg2 = neg_p.astype(jnp.int32).reshape(_NEG_TILES, _ROWS_PER_TILE)
    ci2 = Ci.astype(jnp.int32).reshape(_NEG_TILES, _ROWS_PER_TILE)
    pidx = jnp.concatenate(
        [jnp.concatenate([neg2, ci2], axis=1).reshape(-1),
         jnp.broadcast_to(poii.astype(jnp.int32), (_ROWS_PER_TILE,)),
         jnp.broadcast_to(userid.astype(jnp.int32), (_ROWS_PER_TILE,))])
    packed = _make_sc_gather()(pidx, UserPreference, PoiPreference)
    return _finish(packed)[0, 0]


# speedup vs baseline: 1.2703x; 1.0387x over previous
import functools

import jax
import jax.numpy as jnp
from jax import lax
from jax.experimental import pallas as pl
from jax.experimental.pallas import tpu as pltpu
from jax.experimental.pallas import tpu_sc as plsc

_NEG = 200
_CI = 200
_D = 128
_ROWS_PER_TILE = 8
_NEG_TILES = _NEG // _ROWS_PER_TILE
_PI_TILE = _NEG_TILES
_U_TILE = _NEG_TILES + 1
_CSUM_BASE = _NEG + _ROWS_PER_TILE
_U_ROW = _CSUM_BASE + _NEG_TILES
_PACK_ROWS = 240


@functools.cache
def _make_sc_gather():
    @functools.partial(
        pl.kernel,
        out_type=jax.ShapeDtypeStruct((_PACK_ROWS, _D), jnp.float32),
        mesh=plsc.VectorSubcoreMesh(core_axis_name="c", subcore_axis_name="s"),
        scratch_types=[
            pltpu.VMEM((16,), jnp.int32),
            pltpu.VMEM((16, _D), jnp.float32),
            pltpu.VMEM((1, _D), jnp.float32),
            pltpu.SemaphoreType.DMA,
            pltpu.SemaphoreType.DMA,
        ],
    )
    def _sc_gather(uid_hbm, pid_hbm, ci_hbm, neg_hbm, up_hbm, pp_hbm, out_hbm,
                   idx16, rows16, srow, sem1, sem2):
        wid = lax.axis_index("s") * 2 + lax.axis_index("c")

        @pl.when(wid < _NEG_TILES)
        def _():
            base = wid * _ROWS_PER_TILE
            c1 = pltpu.async_copy(neg_hbm.at[pl.ds(base, 8)],
                                  idx16.at[pl.ds(0, 8)], sem1)
            c2 = pltpu.async_copy(ci_hbm.at[pl.ds(base, 8)],
                                  idx16.at[pl.ds(8, 8)], sem2)
            c1.wait()
            c2.wait()
            pltpu.async_copy(pp_hbm.at[idx16], rows16, sem1).wait()
            w1 = pltpu.async_copy(rows16.at[pl.ds(0, _ROWS_PER_TILE)],
                                  out_hbm.at[pl.ds(base, _ROWS_PER_TILE)], sem2)
            for c in range(_D // 16):
                sl = pl.ds(c * 16, 16)
                acc = rows16[8, sl]
                for r in range(9, 16):
                    acc = acc + rows16[r, sl]
                srow[0, sl] = acc
            w2 = pltpu.async_copy(srow, out_hbm.at[pl.ds(_CSUM_BASE + wid, 1)],
                                  sem1)
            w1.wait()
            w2.wait()

        @pl.when(wid == _PI_TILE)
        def _():
            pltpu.sync_copy(pid_hbm, idx16.at[pl.ds(0, 1)])
            i = idx16[...][0]
            pltpu.sync_copy(pp_hbm.at[pl.ds(i, 1)], rows16.at[pl.ds(0, 1)])
            pltpu.sync_copy(rows16.at[pl.ds(0, 1)], out_hbm.at[pl.ds(_NEG, 1)])

        @pl.when(wid == _U_TILE)
        def _():
            pltpu.sync_copy(uid_hbm, idx16.at[pl.ds(0, 1)])
            i = idx16[...][0]
            pltpu.sync_copy(up_hbm.at[pl.ds(i, 1)], rows16.at[pl.ds(0, 1)])
            pltpu.sync_copy(rows16.at[pl.ds(0, 1)], out_hbm.at[pl.ds(_U_ROW, 1)])

    return _sc_gather


def _finish_body(x_ref, o_ref):
    neg = x_ref[0:_NEG, :]
    pi = x_ref[_NEG:_NEG + 1, :]
    u = x_ref[_U_ROW:_U_ROW + 1, :]
    csum = jnp.sum(x_ref[_CSUM_BASE:_CSUM_BASE + _NEG_TILES, :],
                   axis=0, keepdims=True)

    s = jnp.sum(u * pi)
    t = jnp.sum(csum * pi) / float(_CI)
    a = jnp.sum(neg * u, axis=1, keepdims=True)
    b = jnp.sum(neg * csum, axis=1, keepdims=True) / float(_CI)

    score = jax.nn.sigmoid(s) * jax.nn.sigmoid(t)
    neg_score = jax.nn.sigmoid(a) * jax.nn.sigmoid(b)
    loss = -(jnp.log(score) + jnp.sum(jnp.log(1.0 - neg_score)))
    o_ref[0, 0] = loss


_finish = pl.pallas_call(
    _finish_body,
    out_shape=jax.ShapeDtypeStruct((1, 1), jnp.float32),
    out_specs=pl.BlockSpec(memory_space=pltpu.SMEM),
)


def kernel(userid, poii, Ci, neg_p, UserPreference, PoiPreference):
    packed = _make_sc_gather()(
        userid.astype(jnp.int32), poii.astype(jnp.int32),
        Ci.astype(jnp.int32), neg_p.astype(jnp.int32),
        UserPreference, PoiPreference)
    return _finish(packed)[0, 0]


# speedup vs baseline: 1.2723x; 1.0016x over previous
import functools

import jax
import jax.numpy as jnp
from jax import lax
from jax.experimental import pallas as pl
from jax.experimental.pallas import tpu as pltpu
from jax.experimental.pallas import tpu_sc as plsc

_NEG = 200
_CI = 200
_D = 128
_ROWS_PER_TILE = 8
_NEG_TILES = _NEG // _ROWS_PER_TILE
_PI_TILE = _NEG_TILES
_U_TILE = _NEG_TILES + 1
_CSUM_BASE = _NEG + _ROWS_PER_TILE
_U_ROW = _CSUM_BASE + _NEG_TILES
_PACK_ROWS = 240


@functools.cache
def _make_sc_gather():
    @functools.partial(
        pl.kernel,
        out_type=jax.ShapeDtypeStruct((_PACK_ROWS, _D), jnp.float32),
        mesh=plsc.VectorSubcoreMesh(core_axis_name="c", subcore_axis_name="s"),
        scratch_types=[
            pltpu.VMEM((16,), jnp.int32),
            pltpu.VMEM((16, _D), jnp.float32),
            pltpu.VMEM((1, _D), jnp.float32),
            pltpu.SemaphoreType.DMA,
            pltpu.SemaphoreType.DMA,
        ],
    )
    def _sc_gather(uid_hbm, pid_hbm, ci_hbm, neg_hbm, up_hbm, pp_hbm, out_hbm,
                   idx16, rows16, srow, sem1, sem2):
        wid = lax.axis_index("s") * 2 + lax.axis_index("c")

        @pl.when(wid < _NEG_TILES)
        def _():
            base = wid * _ROWS_PER_TILE
            c1 = pltpu.async_copy(neg_hbm.at[pl.ds(base, 8)],
                                  idx16.at[pl.ds(0, 8)], sem1)
            c2 = pltpu.async_copy(ci_hbm.at[pl.ds(base, 8)],
                                  idx16.at[pl.ds(8, 8)], sem2)
            c1.wait()
            c2.wait()
            pltpu.async_copy(pp_hbm.at[idx16], rows16, sem1).wait()
            w1 = pltpu.async_copy(rows16.at[pl.ds(0, _ROWS_PER_TILE)],
                                  out_hbm.at[pl.ds(base, _ROWS_PER_TILE)], sem2)
            for c in range(_D // 16):
                sl = pl.ds(c * 16, 16)
                acc = rows16[8, sl]
                for r in range(9, 16):
                    acc = acc + rows16[r, sl]
                srow[0, sl] = acc
            w2 = pltpu.async_copy(srow, out_hbm.at[pl.ds(_CSUM_BASE + wid, 1)],
                                  sem1)
            w1.wait()
            w2.wait()

        @pl.when(wid == _PI_TILE)
        def _():
            pltpu.sync_copy(pid_hbm, idx16.at[pl.ds(0, 1)])
            i = idx16[...][0]
            pltpu.sync_copy(pp_hbm.at[pl.ds(i, 1)], out_hbm.at[pl.ds(_NEG, 1)])

        @pl.when(wid == _U_TILE)
        def _():
            pltpu.sync_copy(uid_hbm, idx16.at[pl.ds(0, 1)])
            i = idx16[...][0]
            pltpu.sync_copy(up_hbm.at[pl.ds(i, 1)], out_hbm.at[pl.ds(_U_ROW, 1)])

    return _sc_gather


def _finish_body(x_ref, o_ref):
    neg = x_ref[0:_NEG, :]
    pi = x_ref[_NEG:_NEG + 1, :]
    u = x_ref[_U_ROW:_U_ROW + 1, :]
    csum = jnp.sum(x_ref[_CSUM_BASE:_CSUM_BASE + _NEG_TILES, :],
                   axis=0, keepdims=True)

    s = jnp.sum(u * pi)
    t = jnp.sum(csum * pi) / float(_CI)
    a = jnp.sum(neg * u, axis=1, keepdims=True)
    b = jnp.sum(neg * csum, axis=1, keepdims=True) / float(_CI)

    score = jax.nn.sigmoid(s) * jax.nn.sigmoid(t)
    neg_score = jax.nn.sigmoid(a) * jax.nn.sigmoid(b)
    loss = -(jnp.log(score) + jnp.sum(jnp.log(1.0 - neg_score)))
    o_ref[0, 0] = loss


_finish = pl.pallas_call(
    _finish_body,
    out_shape=jax.ShapeDtypeStruct((1, 1), jnp.float32),
    out_specs=pl.BlockSpec(memory_space=pltpu.SMEM),
)


def kernel(userid, poii, Ci, neg_p, UserPreference, PoiPreference):
    packed = _make_sc_gather()(
        userid.astype(jnp.int32), poii.astype(jnp.int32),
        Ci.astype(jnp.int32), neg_p.astype(jnp.int32),
        UserPreference, PoiPreference)
    return _finish(packed)[0, 0]


# speedup vs baseline: 1.2725x; 1.0001x over previous
import functools

import jax
import jax.numpy as jnp
from jax import lax
from jax.experimental import pallas as pl
from jax.experimental.pallas import tpu as pltpu
from jax.experimental.pallas import tpu_sc as plsc

_NEG = 200
_CI = 200
_D = 128
_ROWS_PER_TILE = 8
_NEG_TILES = _NEG // _ROWS_PER_TILE
_PI_TILE = _NEG_TILES
_U_TILE = _NEG_TILES + 1
_CSUM_BASE = _NEG + _ROWS_PER_TILE
_U_ROW = _CSUM_BASE + _NEG_TILES
_PACK_ROWS = 240


@functools.cache
def _make_sc_gather():
    @functools.partial(
        pl.kernel,
        out_type=jax.ShapeDtypeStruct((_PACK_ROWS, _D), jnp.float32),
        mesh=plsc.VectorSubcoreMesh(core_axis_name="c", subcore_axis_name="s"),
        scratch_types=[
            pltpu.VMEM((16,), jnp.int32),
            pltpu.VMEM((16, _D), jnp.float32),
            pltpu.VMEM((1, _D), jnp.float32),
            pltpu.SemaphoreType.DMA,
            pltpu.SemaphoreType.DMA,
        ],
    )
    def _sc_gather(uid_hbm, pid_hbm, ci_hbm, neg_hbm, up_hbm, pp_hbm, out_hbm,
                   idx16, rows16, srow, sem1, sem2):
        wid = lax.axis_index("s") * 2 + lax.axis_index("c")

        @pl.when(wid < _NEG_TILES)
        def _():
            base = wid * _ROWS_PER_TILE
            c1 = pltpu.async_copy(neg_hbm.at[pl.ds(base, 8)],
                                  idx16.at[pl.ds(0, 8)], sem1)
            c2 = pltpu.async_copy(ci_hbm.at[pl.ds(base, 8)],
                                  idx16.at[pl.ds(8, 8)], sem2)
            c1.wait()
            c2.wait()
            pltpu.async_copy(pp_hbm.at[idx16], rows16, sem1).wait()
            w1 = pltpu.async_copy(rows16.at[pl.ds(0, _ROWS_PER_TILE)],
                                  out_hbm.at[pl.ds(base, _ROWS_PER_TILE)], sem2)
            for c in range(_D // 16):
                sl = pl.ds(c * 16, 16)
                acc = rows16[8, sl]
                for r in range(9, 16):
                    acc = acc + rows16[r, sl]
                srow[0, sl] = acc
            w2 = pltpu.async_copy(srow, out_hbm.at[pl.ds(_CSUM_BASE + wid, 1)],
                                  sem1)
            w1.wait()
            w2.wait()

        @pl.when(wid == _PI_TILE)
        def _():
            pltpu.sync_copy(pid_hbm, idx16.at[pl.ds(0, 1)])
            i = idx16[...][0]
            pltpu.sync_copy(pp_hbm.at[pl.ds(i, 1)], out_hbm.at[pl.ds(_NEG, 1)])

        @pl.when(wid == _U_TILE)
        def _():
            pltpu.sync_copy(uid_hbm, idx16.at[pl.ds(0, 1)])
            i = idx16[...][0]
            pltpu.sync_copy(up_hbm.at[pl.ds(i, 1)], out_hbm.at[pl.ds(_U_ROW, 1)])

    return _sc_gather


def _finish_body(x_ref, o_ref):
    neg = x_ref[0:_NEG, :]
    pi = x_ref[_NEG:_NEG + 1, :]
    u = x_ref[_U_ROW:_U_ROW + 1, :]
    csum = jnp.sum(x_ref[_CSUM_BASE:_CSUM_BASE + _NEG_TILES, :],
                   axis=0, keepdims=True)

    s = jnp.sum(u * pi)
    t = jnp.sum(csum * pi) / float(_CI)
    dn = (((1,), (1,)), ((), ()))
    a = lax.dot_general(u, neg, dn)
    b = lax.dot_general(csum, neg, dn) / float(_CI)

    score = jax.nn.sigmoid(s) * jax.nn.sigmoid(t)
    neg_score = jax.nn.sigmoid(a) * jax.nn.sigmoid(b)
    loss = -(jnp.log(score) + jnp.sum(jnp.log(1.0 - neg_score)))
    o_ref[0, 0] = loss


_finish = pl.pallas_call(
    _finish_body,
    out_shape=jax.ShapeDtypeStruct((1, 1), jnp.float32),
    out_specs=pl.BlockSpec(memory_space=pltpu.SMEM),
)


def kernel(userid, poii, Ci, neg_p, UserPreference, PoiPreference):
    packed = _make_sc_gather()(
        userid.astype(jnp.int32), poii.astype(jnp.int32),
        Ci.astype(jnp.int32), neg_p.astype(jnp.int32),
        UserPreference, PoiPreference)
    return _finish(packed)[0, 0]


# speedup vs baseline: 1.3484x; 1.0597x over previous
import functools

import jax
import jax.numpy as jnp
from jax import lax
from jax.experimental import pallas as pl
from jax.experimental.pallas import tpu as pltpu
from jax.experimental.pallas import tpu_sc as plsc

_NEG = 200
_CI = 200
_D = 128
_GATHER_TILES = 13
_PI_TILE = 13
_U_TILE = 14
_PI_ROW = 200
_CSUM_BASE = 201
_U_ROW = 214
_PACK_ROWS = 216
_LAST_OFF = 184


@functools.cache
def _make_sc_gather():
    @functools.partial(
        pl.kernel,
        out_type=jax.ShapeDtypeStruct((_PACK_ROWS, _D), jnp.float32),
        mesh=plsc.VectorSubcoreMesh(core_axis_name="c", subcore_axis_name="s",
                                    num_cores=1),
        scratch_types=[
            pltpu.VMEM((32,), jnp.int32),
            pltpu.VMEM((32, _D), jnp.float32),
            pltpu.VMEM((1, _D), jnp.float32),
            pltpu.SemaphoreType.DMA,
            pltpu.SemaphoreType.DMA,
        ],
    )
    def _sc_gather(uid_hbm, pid_hbm, ci_hbm, neg_hbm, up_hbm, pp_hbm, out_hbm,
                   idx32, rows32, srow, sem1, sem2):
        s = lax.axis_index("s")

        @pl.when(s < _GATHER_TILES)
        def _():
            base = jnp.minimum(s * 16, _LAST_OFF)
            c1 = pltpu.async_copy(neg_hbm.at[pl.ds(base, 16)],
                                  idx32.at[pl.ds(0, 16)], sem1)
            c2 = pltpu.async_copy(ci_hbm.at[pl.ds(base, 16)],
                                  idx32.at[pl.ds(16, 16)], sem2)
            c1.wait()
            c2.wait()
            pltpu.async_copy(pp_hbm.at[idx32], rows32, sem1).wait()
            w1 = pltpu.async_copy(rows32.at[pl.ds(0, 16)],
                                  out_hbm.at[pl.ds(base, 16)], sem2)
            lo_w = jnp.broadcast_to(
                jnp.where(s < _GATHER_TILES - 1, 1.0, 0.0).astype(jnp.float32),
                (16,))
            for c in range(_D // 16):
                sl = pl.ds(c * 16, 16)
                lo = rows32[16, sl]
                for r in range(17, 24):
                    lo = lo + rows32[r, sl]
                hi = rows32[24, sl]
                for r in range(25, 32):
                    hi = hi + rows32[r, sl]
                srow[0, sl] = lo * lo_w + hi
            w2 = pltpu.async_copy(srow, out_hbm.at[pl.ds(_CSUM_BASE + s, 1)],
                                  sem1)
            w1.wait()
            w2.wait()

        @pl.when(s == _PI_TILE)
        def _():
            pltpu.sync_copy(pid_hbm, idx32.at[pl.ds(0, 1)])
            i = idx32[...][0]
            pltpu.sync_copy(pp_hbm.at[pl.ds(i, 1)], out_hbm.at[pl.ds(_PI_ROW, 1)])

        @pl.when(s == _U_TILE)
        def _():
            pltpu.sync_copy(uid_hbm, idx32.at[pl.ds(0, 1)])
            i = idx32[...][0]
            pltpu.sync_copy(up_hbm.at[pl.ds(i, 1)], out_hbm.at[pl.ds(_U_ROW, 1)])

    return _sc_gather


def _finish_body(x_ref, o_ref):
    neg = x_ref[0:_NEG, :]
    pi = x_ref[_PI_ROW:_PI_ROW + 1, :]
    u = x_ref[_U_ROW:_U_ROW + 1, :]
    csum = jnp.sum(x_ref[_CSUM_BASE:_CSUM_BASE + _GATHER_TILES, :],
                   axis=0, keepdims=True)

    s = jnp.sum(u * pi)
    t = jnp.sum(csum * pi) / float(_CI)
    dn = (((1,), (1,)), ((), ()))
    a = lax.dot_general(u, neg, dn)
    b = lax.dot_general(csum, neg, dn) / float(_CI)

    score = jax.nn.sigmoid(s) * jax.nn.sigmoid(t)
    neg_score = jax.nn.sigmoid(a) * jax.nn.sigmoid(b)
    loss = -(jnp.log(score) + jnp.sum(jnp.log(1.0 - neg_score)))
    o_ref[0, 0] = loss


_finish = pl.pallas_call(
    _finish_body,
    out_shape=jax.ShapeDtypeStruct((1, 1), jnp.float32),
    out_specs=pl.BlockSpec(memory_space=pltpu.SMEM),
)


def kernel(userid, poii, Ci, neg_p, UserPreference, PoiPreference):
    packed = _make_sc_gather()(
        userid.astype(jnp.int32), poii.astype(jnp.int32),
        Ci.astype(jnp.int32), neg_p.astype(jnp.int32),
        UserPreference, PoiPreference)
    return _finish(packed)[0, 0]


# speedup vs baseline: 1.4624x; 1.0845x over previous
import functools

import jax
import jax.numpy as jnp
from jax import lax
from jax.experimental import pallas as pl
from jax.experimental.pallas import tpu as pltpu
from jax.experimental.pallas import tpu_sc as plsc

_NEG = 200
_CI = 200
_D = 128
_GATHER_TILES = 13
_PI_TILE = 13
_U_TILE = 14
_PI_ROW = 200
_CSUM_BASE = 201
_U_ROW = 214
_PACK_ROWS = 216
_LAST_OFF = 184


@functools.cache
def _make_sc_gather():
    @functools.partial(
        pl.kernel,
        out_type=jax.ShapeDtypeStruct((_PACK_ROWS, _D), jnp.float32),
        mesh=plsc.VectorSubcoreMesh(core_axis_name="c", subcore_axis_name="s",
                                    num_cores=1),
        scratch_types=[
            pltpu.VMEM((32,), jnp.int32),
            pltpu.VMEM((32, _D), jnp.float32),
            pltpu.VMEM((1, _D), jnp.float32),
            pltpu.SemaphoreType.DMA,
            pltpu.SemaphoreType.DMA,
        ],
    )
    def _sc_gather(uid_hbm, pid_hbm, ci_hbm, neg_hbm, up_hbm, pp_hbm, out_hbm,
                   idx32, rows32, srow, sem1, sem2):
        s = lax.axis_index("s")

        @pl.when(s < _GATHER_TILES)
        def _():
            base = jnp.minimum(s * 16, _LAST_OFF)
            c1 = pltpu.async_copy(neg_hbm.at[pl.ds(base, 16)],
                                  idx32.at[pl.ds(0, 16)], sem1)
            c2 = pltpu.async_copy(ci_hbm.at[pl.ds(base, 16)],
                                  idx32.at[pl.ds(16, 16)], sem2)
            c1.wait()
            g1 = pltpu.async_copy(pp_hbm.at[idx32.at[pl.ds(0, 16)]],
                                  rows32.at[pl.ds(0, 16)], sem1)
            c2.wait()
            g2 = pltpu.async_copy(pp_hbm.at[idx32.at[pl.ds(16, 16)]],
                                  rows32.at[pl.ds(16, 16)], sem2)
            g1.wait()
            w1 = pltpu.async_copy(rows32.at[pl.ds(0, 16)],
                                  out_hbm.at[pl.ds(base, 16)], sem1)
            g2.wait()
            lo_w = jnp.broadcast_to(
                jnp.where(s < _GATHER_TILES - 1, 1.0, 0.0).astype(jnp.float32),
                (16,))
            for c in range(_D // 16):
                sl = pl.ds(c * 16, 16)
                lo = rows32[16, sl]
                for r in range(17, 24):
                    lo = lo + rows32[r, sl]
                hi = rows32[24, sl]
                for r in range(25, 32):
                    hi = hi + rows32[r, sl]
                srow[0, sl] = lo * lo_w + hi
            w2 = pltpu.async_copy(srow, out_hbm.at[pl.ds(_CSUM_BASE + s, 1)],
                                  sem2)
            w1.wait()
            w2.wait()

        @pl.when(s == _PI_TILE)
        def _():
            pltpu.sync_copy(pid_hbm, idx32.at[pl.ds(0, 1)])
            i = idx32[...][0]
            pltpu.sync_copy(pp_hbm.at[pl.ds(i, 1)], out_hbm.at[pl.ds(_PI_ROW, 1)])

        @pl.when(s == _U_TILE)
        def _():
            pltpu.sync_copy(uid_hbm, idx32.at[pl.ds(0, 1)])
            i = idx32[...][0]
            pltpu.sync_copy(up_hbm.at[pl.ds(i, 1)], out_hbm.at[pl.ds(_U_ROW, 1)])

    return _sc_gather


def _finish_body(x_ref, o_ref):
    neg = x_ref[0:_NEG, :]
    pi = x_ref[_PI_ROW:_PI_ROW + 1, :]
    u = x_ref[_U_ROW:_U_ROW + 1, :]
    csum = jnp.sum(x_ref[_CSUM_BASE:_CSUM_BASE + _GATHER_TILES, :],
                   axis=0, keepdims=True)

    s = jnp.sum(u * pi)
    t = jnp.sum(csum * pi) / float(_CI)
    dn = (((1,), (1,)), ((), ()))
    a = lax.dot_general(u, neg, dn)
    b = lax.dot_general(csum, neg, dn) / float(_CI)

    score = jax.nn.sigmoid(s) * jax.nn.sigmoid(t)
    neg_score = jax.nn.sigmoid(a) * jax.nn.sigmoid(b)
    loss = -(jnp.log(score) + jnp.sum(jnp.log(1.0 - neg_score)))
    o_ref[0, 0] = loss


_finish = pl.pallas_call(
    _finish_body,
    out_shape=jax.ShapeDtypeStruct((1, 1), jnp.float32),
    out_specs=pl.BlockSpec(memory_space=pltpu.SMEM),
)


def kernel(userid, poii, Ci, neg_p, UserPreference, PoiPreference):
    packed = _make_sc_gather()(
        userid.astype(jnp.int32), poii.astype(jnp.int32),
        Ci.astype(jnp.int32), neg_p.astype(jnp.int32),
        UserPreference, PoiPreference)
    return _finish(packed)[0, 0]
